# baseline (speedup 1.0000x reference)
import functools

import jax
import jax.numpy as jnp
from jax import lax
from jax.experimental import pallas as pl
from jax.experimental.pallas import tpu as pltpu
from jax.experimental.pallas import tpu_sc as plsc

N = 10000
E = 320000
IN_FEATS = 128
H_FEATS = 64
NUM_CLASSES = 40

NC = 2
NS = 16
NW = NC * NS
OWN = 320
NPAD = NW * OWN
ECH = 6400
NECH = E // ECH
SCAP = ECH + 128
BLK = 128


def _tc_proj(x, W, al, ar, one_col):
  n, _ = x.shape
  d = W.shape[1]

  def body(x_ref, w_ref, al_ref, ar_ref, oh, oel, oer, ogm):
    h = jnp.dot(x_ref[...], w_ref[...], preferred_element_type=jnp.float32)
    el = jnp.sum(h * al_ref[...], axis=1, keepdims=True)
    er = jnp.sum(h * ar_ref[...], axis=1, keepdims=True)
    col = lax.broadcasted_iota(jnp.int32, (n, d), 1)
    oh[...] = jnp.where(col == one_col, 1.0, h)
    oel[...] = el
    oer[...] = er
    ogm[...] = jnp.full((8, 128), jnp.max(el), dtype=jnp.float32)

  return pl.pallas_call(
      body,
      out_shape=(
          jax.ShapeDtypeStruct((n, d), jnp.float32),
          jax.ShapeDtypeStruct((n, 1), jnp.float32),
          jax.ShapeDtypeStruct((n, 1), jnp.float32),
          jax.ShapeDtypeStruct((8, 128), jnp.float32),
      ),
  )(x, W, al, ar)


def _tc_combine_proj(acc, dw, b, W, al, ar, one_col):
  d2 = W.shape[1]

  def body(p_ref, b_ref, w_ref, al_ref, ar_ref, oh, oel, oer, ogm):
    s = p_ref[:N, :]
    x = s[:, :dw] / (s[:, dw:dw + 1] + 1e-9) + b_ref[...]
    x = jnp.where(x > 0, x, jnp.exp(jnp.minimum(x, 0.0)) - 1.0)
    h = jnp.dot(x, w_ref[...], preferred_element_type=jnp.float32)
    el = jnp.sum(h * al_ref[...], axis=1, keepdims=True)
    er = jnp.sum(h * ar_ref[...], axis=1, keepdims=True)
    col = lax.broadcasted_iota(jnp.int32, (N, d2), 1)
    oh[...] = jnp.where(col == one_col, 1.0, h)
    oel[...] = el
    oer[...] = er
    ogm[...] = jnp.full((8, 128), jnp.max(el), dtype=jnp.float32)

  return pl.pallas_call(
      body,
      out_shape=(
          jax.ShapeDtypeStruct((N, d2), jnp.float32),
          jax.ShapeDtypeStruct((N, 1), jnp.float32),
          jax.ShapeDtypeStruct((N, 1), jnp.float32),
          jax.ShapeDtypeStruct((8, 128), jnp.float32),
      ),
  )(acc, b, W, al, ar)


def _tc_finish(acc, dw, b):
  def body(p_ref, b_ref, o):
    s = p_ref[:N, :]
    o[...] = s[:, :dw] / (s[:, dw:dw + 1] + 1e-9) + b_ref[...]

  return pl.pallas_call(
      body,
      out_shape=jax.ShapeDtypeStruct((N, dw), jnp.float32),
  )(acc, b)


def _make_sc_gat(D):
  NQ = D // 16
  mesh = plsc.VectorSubcoreMesh(
      core_axis_name="c", subcore_axis_name="s", num_cores=NC, num_subcores=NS)

  @functools.partial(
      pl.kernel,
      out_type=jax.ShapeDtypeStruct((NPAD, D), jnp.float32),
      mesh=mesh,
      compiler_params=pltpu.CompilerParams(
          needs_layout_passes=False, use_tc_tiling_on_sc=False),
      scratch_types=[
          pltpu.VMEM((ECH,), jnp.int32),
          pltpu.VMEM((ECH,), jnp.int32),
          pltpu.VMEM((ECH,), jnp.int32),
          pltpu.VMEM((ECH,), jnp.int32),
          pltpu.VMEM((N,), jnp.float32),
          pltpu.VMEM((OWN,), jnp.float32),
          pltpu.VMEM((16,), jnp.float32),
          pltpu.VMEM((SCAP,), jnp.int32),
          pltpu.VMEM((SCAP,), jnp.int32),
          pltpu.VMEM((BLK, D), jnp.float32),
          pltpu.VMEM((BLK, D), jnp.float32),
          pltpu.VMEM((BLK,), jnp.float32),
          pltpu.VMEM((BLK,), jnp.int32),
          pltpu.VMEM((OWN, D), jnp.float32),
          pltpu.SemaphoreType.DMA,
          pltpu.SemaphoreType.DMA,
          pltpu.SemaphoreType.DMA,
          pltpu.SemaphoreType.DMA,
      ],
  )
  def sc_gat(h_hbm, el_hbm, er_hbm, gm_hbm, src_hbm, dst_hbm, out_hbm,
             sb0, db0, sb1, db1, el_v, erl_v, gm_v, stg_s, stg_d,
             rows0, rows1, w_v, dl_v, acc_v,
             sem_g0, sem_g1, sem_e0, sem_e1):
    c = lax.axis_index("c")
    s = lax.axis_index("s")
    wid = c * NS + s
    lo = wid * OWN

    zf = jnp.zeros((16,), jnp.float32)
    zi = jnp.zeros((16,), jnp.int32)

    def zr(r, carry):
      for q in range(NQ):
        acc_v[r, pl.ds(q * 16, 16)] = zf
      return carry

    lax.fori_loop(0, OWN, zr, 0)

    def zs(g, carry):
      stg_s[pl.ds(g * 16, 16)] = zi
      stg_d[pl.ds(g * 16, 16)] = zi
      return carry

    lax.fori_loop(0, SCAP // 16, zs, 0)

    pltpu.sync_copy(el_hbm, el_v)
    pltpu.sync_copy(er_hbm.at[pl.ds(lo, OWN)], erl_v)
    pltpu.sync_copy(gm_hbm, gm_v)
    pltpu.async_copy(src_hbm.at[pl.ds(0, ECH)], sb0, sem_e0)
    pltpu.async_copy(dst_hbm.at[pl.ds(0, ECH)], db0, sem_e0)
    pltpu.async_copy(src_hbm.at[pl.ds(ECH, ECH)], sb1, sem_e1)
    pltpu.async_copy(dst_hbm.at[pl.ds(ECH, ECH)], db1, sem_e1)

    iota16 = lax.iota(jnp.int32, 16)
    BUFS = ((rows0, sem_g0), (rows1, sem_g1))

    def fire(j, bufs):
      rows, sem = bufs
      pltpu.async_copy(h_hbm.at[stg_s.at[pl.ds(j * BLK, BLK)]], rows, sem)

    def wait_process(j, bufs, cnt_valid):
      rows, sem = bufs
      pltpu.make_async_copy(h_hbm.at[stg_s.at[pl.ds(j * BLK, BLK)]], rows,
                            sem).wait()

      def grp(g, carry):
        si = stg_s[pl.ds(j * BLK + g * 16, 16)]
        di = stg_d[pl.ds(j * BLK + g * 16, 16)]
        dl = jnp.clip(di - lo, 0, OWN - 1)
        els = plsc.load_gather(el_v, [si])
        erd = plsc.load_gather(erl_v, [dl])
        e = els + erd
        e = jnp.maximum(e, 0.2 * e)
        t = gm_v[...] + erd
        m = jnp.maximum(t, 0.2 * t)
        w = jnp.exp(e - m)
        w = jnp.where(g * 16 + iota16 < cnt_valid, w, 0.0)
        w_v[pl.ds(g * 16, 16)] = w
        dl_v[pl.ds(g * 16, 16)] = dl
        return carry

      lax.fori_loop(0, BLK // 16, grp, 0, unroll=2)

      def accg(g, carry):
        wg = w_v[pl.ds(g * 16, 16)]
        dlg = dl_v[pl.ds(g * 16, 16)]
        for k in range(16):
          wc = wg[k]
          dl = dlg[k]
          r = g * 16 + k
          for q in range(NQ):
            plsc.addupdate(acc_v.at[dl, pl.ds(q * 16, 16)],
                           rows[r, pl.ds(q * 16, 16)] * wc)
        return carry

      lax.fori_loop(0, BLK // 16, accg, 0)

    def process_blocks(cnt):
      nb = cnt // BLK

      @pl.when(nb > 0)
      def _():
        fire(0, BUFS[0])

      def pblk(j, carry):
        @pl.when(jnp.logical_and(j + 1 < nb, j % 2 == 0))
        def _():
          fire(j + 1, BUFS[1])

        @pl.when(jnp.logical_and(j + 1 < nb, j % 2 == 1))
        def _():
          fire(j + 1, BUFS[0])

        @pl.when(j % 2 == 0)
        def _():
          wait_process(j, BUFS[0], BLK)

        @pl.when(j % 2 == 1)
        def _():
          wait_process(j, BUFS[1], BLK)

        return carry

      lax.fori_loop(0, nb, pblk, 0)

      base = nb * BLK
      for g in range(BLK // 16):
        stg_s[pl.ds(g * 16, 16)] = stg_s[pl.ds(base + g * 16, 16)]
        stg_d[pl.ds(g * 16, 16)] = stg_d[pl.ds(base + g * 16, 16)]
      return cnt - base

    def pair(i, cnt):
      for p, (sb, db, sem_e) in enumerate(
          ((sb0, db0, sem_e0), (sb1, db1, sem_e1))):
        pltpu.make_async_copy(src_hbm.at[pl.ds(0, ECH)], sb, sem_e).wait()
        pltpu.make_async_copy(dst_hbm.at[pl.ds(0, ECH)], db, sem_e).wait()

        cntv = jnp.full((16,), cnt, dtype=jnp.int32)

        def scanw(wi, cntv):
          s16 = sb[pl.ds(wi * 16, 16)]
          d16 = db[pl.ds(wi * 16, 16)]
          msk = (d16 >= lo) & (d16 < lo + OWN)
          rank = plsc.cumsum(jnp.where(msk, 1, 0)) - 1
          idx = cntv + rank
          plsc.store_scatter(stg_s, [idx], s16, mask=msk)
          plsc.store_scatter(stg_d, [idx], d16, mask=msk)
          return cntv + plsc.all_reduce_population_count(msk)

        cntv = lax.fori_loop(0, ECH // 16, scanw, cntv, unroll=4)
        cnt = cntv[0]

        nxt = (2 * i + p + 2) * ECH

        @pl.when(i < NECH // 2 - 1)
        def _start_next():
          pltpu.async_copy(src_hbm.at[pl.ds(nxt, ECH)], sb, sem_e)
          pltpu.async_copy(dst_hbm.at[pl.ds(nxt, ECH)], db, sem_e)

        cnt = process_blocks(cnt)
      return cnt

    cnt = lax.fori_loop(0, NECH // 2, pair, jnp.int32(0))

    fire(0, BUFS[0])
    wait_process(0, BUFS[0], cnt)

    pltpu.sync_copy(acc_v, out_hbm.at[pl.ds(lo, OWN)])

  return sc_gat


_sc_gat_l1 = _make_sc_gat(80)
_sc_gat_l2 = _make_sc_gat(48)


def kernel(in_feat, edge_index, W1, a_l1, a_r1, b1, W2, a_l2, a_r2, b2):
  f32 = jnp.float32
  src = edge_index[0]
  dst = edge_index[1]

  W1a = jnp.pad(W1, ((0, 0), (0, 16)))
  al1 = jnp.pad(a_l1, (0, 16)).reshape(1, 80)
  ar1 = jnp.pad(a_r1, (0, 16)).reshape(1, 80)
  h1, el1, er1, gm1 = _tc_proj(in_feat, W1a, al1, ar1, 64)
  gm1v = jnp.full((16,), gm1[0, 0], dtype=f32)
  er1p = jnp.pad(er1.reshape(N), (0, NPAD - N))
  acc1 = _sc_gat_l1(h1, el1.reshape(N), er1p, gm1v, src, dst)

  W2a = jnp.pad(W2, ((0, 0), (0, 8)))
  al2 = jnp.pad(a_l2, (0, 8)).reshape(1, 48)
  ar2 = jnp.pad(a_r2, (0, 8)).reshape(1, 48)
  h2, el2, er2, gm2 = _tc_combine_proj(acc1, 64, b1.reshape(1, 64),
                                       W2a, al2, ar2, 40)
  gm2v = jnp.full((16,), gm2[0, 0], dtype=f32)
  er2p = jnp.pad(er2.reshape(N), (0, NPAD - N))
  acc2 = _sc_gat_l2(h2, el2.reshape(N), er2p, gm2v, src, dst)

  return _tc_finish(acc2, 40, b2.reshape(1, 40))


# speedup vs baseline: 1.5430x; 1.5430x over previous
import functools

import jax
import jax.numpy as jnp
from jax import lax
from jax.experimental import pallas as pl
from jax.experimental.pallas import tpu as pltpu
from jax.experimental.pallas import tpu_sc as plsc

N = 10000
E = 320000
IN_FEATS = 128
H_FEATS = 64
NUM_CLASSES = 40

NC = 2
NS = 16
NW = NC * NS
CH = 128
EPT = E // NW
NCHUNK = 80
EPT_PAD = NCHUNK * CH
SLAB = 624
TAIL = N - NS * SLAB


def _tc_proj(x, W, al, ar):
  n, _ = x.shape
  d = W.shape[1]

  def body(x_ref, w_ref, al_ref, ar_ref, oh, oel, oer, ogm):
    h = jnp.dot(x_ref[...], w_ref[...], preferred_element_type=jnp.float32)
    oh[...] = h
    el = jnp.sum(h * al_ref[...], axis=1, keepdims=True)
    er = jnp.sum(h * ar_ref[...], axis=1, keepdims=True)
    oel[...] = el
    oer[...] = er
    ogm[...] = jnp.full((8, 128), jnp.max(el), dtype=jnp.float32)

  return pl.pallas_call(
      body,
      out_shape=(
          jax.ShapeDtypeStruct((n, d), jnp.float32),
          jax.ShapeDtypeStruct((n, 1), jnp.float32),
          jax.ShapeDtypeStruct((n, 1), jnp.float32),
          jax.ShapeDtypeStruct((8, 128), jnp.float32),
      ),
  )(x, W, al, ar)


def _tc_combine_proj(parts, dw, b, W, al, ar):
  n = parts.shape[1]
  d2 = W.shape[1]

  def body(p_ref, b_ref, w_ref, al_ref, ar_ref, oh, oel, oer, ogm):
    s = p_ref[0] + p_ref[1]
    x = s[:, :dw] / (s[:, dw:dw + 1] + 1e-9) + b_ref[...]
    x = jnp.where(x > 0, x, jnp.exp(jnp.minimum(x, 0.0)) - 1.0)
    h = jnp.dot(x, w_ref[...], preferred_element_type=jnp.float32)
    oh[...] = h
    el = jnp.sum(h * al_ref[...], axis=1, keepdims=True)
    er = jnp.sum(h * ar_ref[...], axis=1, keepdims=True)
    oel[...] = el
    oer[...] = er
    ogm[...] = jnp.full((8, 128), jnp.max(el), dtype=jnp.float32)

  return pl.pallas_call(
      body,
      out_shape=(
          jax.ShapeDtypeStruct((n, d2), jnp.float32),
          jax.ShapeDtypeStruct((n, 1), jnp.float32),
          jax.ShapeDtypeStruct((n, 1), jnp.float32),
          jax.ShapeDtypeStruct((8, 128), jnp.float32),
      ),
  )(parts, b, W, al, ar)


def _tc_finish(parts, dw, b):
  n = parts.shape[1]

  def body(p_ref, b_ref, o):
    s = p_ref[0] + p_ref[1]
    o[...] = s[:, :dw] / (s[:, dw:dw + 1] + 1e-9) + b_ref[...]

  return pl.pallas_call(
      body,
      out_shape=jax.ShapeDtypeStruct((n, dw), jnp.float32),
  )(parts, b)


def _make_sc_gat(D, DW):
  NG = -(-DW // 16)
  mesh = plsc.VectorSubcoreMesh(
      core_axis_name="c", subcore_axis_name="s", num_cores=NC, num_subcores=NS)

  @functools.partial(
      pl.kernel,
      out_type=jax.ShapeDtypeStruct((NC, N, D), jnp.float32),
      mesh=mesh,
      compiler_params=pltpu.CompilerParams(
          needs_layout_passes=False, use_tc_tiling_on_sc=False),
      scratch_types=[
          pltpu.VMEM((NCHUNK, CH), jnp.int32),
          pltpu.VMEM((NCHUNK, CH), jnp.int32),
          pltpu.VMEM((N,), jnp.float32),
          pltpu.VMEM((N,), jnp.float32),
          pltpu.VMEM((16,), jnp.float32),
          pltpu.VMEM((CH, D), jnp.float32),
          pltpu.VMEM((CH, D), jnp.float32),
          pltpu.VMEM((CH,), jnp.float32),
          pltpu.VMEM_SHARED((N, D), jnp.float32),
          pltpu.SemaphoreType.DMA,
          pltpu.SemaphoreType.DMA,
          pltpu.SemaphoreType.DMA,
          pltpu.SemaphoreType.DMA,
      ],
  )
  def sc_gat(h_hbm, el_hbm, er_hbm, gm_hbm, srcm_hbm, dstm_hbm, zrow_hbm,
             out_hbm, src_v, dst_v, el_v, er_v, gm_v, rows_a, rows_b, w_v,
             acc_sh, sem_ga, sem_gb, sem_sa, sem_sb):
    c = lax.axis_index("c")
    s = lax.axis_index("s")
    wid = c * NS + s

    pltpu.sync_copy(zrow_hbm, acc_sh.at[pl.ds(s * SLAB, SLAB)])

    @pl.when(s == 0)
    def _zero_tail():
      pltpu.sync_copy(zrow_hbm.at[pl.ds(0, TAIL)],
                      acc_sh.at[pl.ds(NS * SLAB, TAIL)])

    pltpu.sync_copy(el_hbm, el_v)
    pltpu.sync_copy(er_hbm, er_v)
    pltpu.sync_copy(gm_hbm, gm_v)
    pltpu.sync_copy(srcm_hbm.at[wid], src_v)
    pltpu.sync_copy(dstm_hbm.at[wid], dst_v)
    plsc.subcore_barrier()

    def compute_scale(j, rows_v):
      def w_body(g, carry2):
        si = src_v[j, pl.ds(g * 16, 16)]
        di = dst_v[j, pl.ds(g * 16, 16)]
        els = plsc.load_gather(el_v, [si])
        erd = plsc.load_gather(er_v, [di])
        e = els + erd
        e = jnp.maximum(e, 0.2 * e)
        t = gm_v[...] + erd
        m = jnp.maximum(t, 0.2 * t)
        w = jnp.exp(e - m)
        pos = j * CH + g * 16 + lax.iota(jnp.int32, 16)
        w = jnp.where(pos < EPT, w, 0.0)
        w_v[pl.ds(g * 16, 16)] = w
        return carry2

      lax.fori_loop(0, CH // 16, w_body, 0, unroll=2)

      def scale_body(g, carry2):
        wg = w_v[pl.ds(g * 16, 16)]
        base = g * 16
        for k in range(16):
          wc = wg[k]
          i = base + k
          for q in range(NG):
            rows_v[i, pl.ds(q * 16, 16)] = rows_v[i, pl.ds(q * 16, 16)] * wc
        row16 = base + lax.iota(jnp.int32, 16)
        col16 = jnp.full((16,), DW, dtype=jnp.int32)
        plsc.store_scatter(rows_v, [row16, col16], wg)
        return carry2

      lax.fori_loop(0, CH // 16, scale_body, 0)

    def fire_gather(j, rows_v, sem_g):
      pltpu.async_copy(h_hbm.at[src_v.at[j]], rows_v, sem_g)

    def wait_gather(j, rows_v, sem_g):
      pltpu.make_async_copy(h_hbm.at[src_v.at[j]], rows_v, sem_g).wait()

    def fire_scatter(j, rows_v, sem_s):
      pltpu.async_copy(rows_v, acc_sh.at[dst_v.at[j]], sem_s, add=True)

    def wait_scatter(j, rows_v, sem_s):
      pltpu.make_async_copy(rows_v, acc_sh.at[dst_v.at[j]], sem_s).wait()

    fire_gather(0, rows_a, sem_ga)

    def pair_body(i, carry):
      a = 2 * i
      b = 2 * i + 1
      wait_gather(a, rows_a, sem_ga)

      @pl.when(i > 0)
      def _():
        wait_scatter(b - 2, rows_b, sem_sb)

      fire_gather(b, rows_b, sem_gb)
      compute_scale(a, rows_a)
      fire_scatter(a, rows_a, sem_sa)

      wait_gather(b, rows_b, sem_gb)

      @pl.when(i + 1 < NCHUNK // 2)
      def _():
        wait_scatter(a, rows_a, sem_sa)
        fire_gather(a + 2, rows_a, sem_ga)

      compute_scale(b, rows_b)
      fire_scatter(b, rows_b, sem_sb)
      return carry

    lax.fori_loop(0, NCHUNK // 2, pair_body, 0)
    wait_scatter(NCHUNK - 2, rows_a, sem_sa)
    wait_scatter(NCHUNK - 1, rows_b, sem_sb)

    plsc.subcore_barrier()
    pltpu.sync_copy(acc_sh.at[pl.ds(s * SLAB, SLAB)],
                    out_hbm.at[c, pl.ds(s * SLAB, SLAB)])

    @pl.when(s == 0)
    def _out_tail():
      pltpu.sync_copy(acc_sh.at[pl.ds(NS * SLAB, TAIL)],
                      out_hbm.at[c, pl.ds(NS * SLAB, TAIL)])

  return sc_gat


_sc_gat_l1 = _make_sc_gat(72, 64)
_sc_gat_l2 = _make_sc_gat(48, 40)


def kernel(in_feat, edge_index, W1, a_l1, a_r1, b1, W2, a_l2, a_r2, b2):
  f32 = jnp.float32
  src = edge_index[0]
  dst = edge_index[1]
  pad = EPT_PAD - EPT
  srcm = jnp.pad(src.reshape(NW, EPT), ((0, 0), (0, pad))).reshape(
      NW, NCHUNK, CH)
  dstm = jnp.pad(dst.reshape(NW, EPT), ((0, 0), (0, pad))).reshape(
      NW, NCHUNK, CH)

  W1a = jnp.pad(W1, ((0, 0), (0, 8)))
  al1 = jnp.pad(a_l1, (0, 8)).reshape(1, 72)
  ar1 = jnp.pad(a_r1, (0, 8)).reshape(1, 72)
  h1, el1, er1, gm1 = _tc_proj(in_feat, W1a, al1, ar1)
  gm1v = jnp.full((16,), gm1[0, 0], dtype=f32)
  z72 = jnp.zeros((SLAB, 72), dtype=f32)
  parts1 = _sc_gat_l1(h1, el1.reshape(N), er1.reshape(N), gm1v,
                      srcm, dstm, z72)

  W2a = jnp.pad(W2, ((0, 0), (0, 8)))
  al2 = jnp.pad(a_l2, (0, 8)).reshape(1, 48)
  ar2 = jnp.pad(a_r2, (0, 8)).reshape(1, 48)
  h2, el2, er2, gm2 = _tc_combine_proj(parts1, 64, b1.reshape(1, 64),
                                       W2a, al2, ar2)
  gm2v = jnp.full((16,), gm2[0, 0], dtype=f32)
  z48 = jnp.zeros((SLAB, 48), dtype=f32)
  parts2 = _sc_gat_l2(h2, el2.reshape(N), er2.reshape(N), gm2v,
                      srcm, dstm, z48)

  return _tc_finish(parts2, 40, b2.reshape(1, 40))


# speedup vs baseline: 1.5514x; 1.0054x over previous
import functools

import jax
import jax.numpy as jnp
from jax import lax
from jax.experimental import pallas as pl
from jax.experimental.pallas import tpu as pltpu
from jax.experimental.pallas import tpu_sc as plsc

N = 10000
E = 320000
IN_FEATS = 128
H_FEATS = 64
NUM_CLASSES = 40

NC = 2
NS = 16
NW = NC * NS
CH = 128
EPT = E // NW
NCHUNK = -(-EPT // CH)
EPT_PAD = NCHUNK * CH
SLAB = 624
TAIL = N - NS * SLAB


def _tc_proj(x, W, al, ar):
  n, _ = x.shape
  d = W.shape[1]

  def body(x_ref, w_ref, al_ref, ar_ref, oh, oel, oer, ogm):
    h = jnp.dot(x_ref[...], w_ref[...], preferred_element_type=jnp.float32)
    oh[...] = h
    el = jnp.sum(h * al_ref[...], axis=1, keepdims=True)
    er = jnp.sum(h * ar_ref[...], axis=1, keepdims=True)
    oel[...] = el
    oer[...] = er
    ogm[...] = jnp.full((8, 128), jnp.max(el), dtype=jnp.float32)

  return pl.pallas_call(
      body,
      out_shape=(
          jax.ShapeDtypeStruct((n, d), jnp.float32),
          jax.ShapeDtypeStruct((n, 1), jnp.float32),
          jax.ShapeDtypeStruct((n, 1), jnp.float32),
          jax.ShapeDtypeStruct((8, 128), jnp.float32),
      ),
  )(x, W, al, ar)


def _tc_combine_proj(parts, dw, b, W, al, ar):
  n = parts.shape[1]
  d2 = W.shape[1]

  def body(p_ref, b_ref, w_ref, al_ref, ar_ref, oh, oel, oer, ogm):
    s = p_ref[0] + p_ref[1]
    x = s[:, :dw] / (s[:, dw:dw + 1] + 1e-9) + b_ref[...]
    x = jnp.where(x > 0, x, jnp.exp(jnp.minimum(x, 0.0)) - 1.0)
    h = jnp.dot(x, w_ref[...], preferred_element_type=jnp.float32)
    oh[...] = h
    el = jnp.sum(h * al_ref[...], axis=1, keepdims=True)
    er = jnp.sum(h * ar_ref[...], axis=1, keepdims=True)
    oel[...] = el
    oer[...] = er
    ogm[...] = jnp.full((8, 128), jnp.max(el), dtype=jnp.float32)

  return pl.pallas_call(
      body,
      out_shape=(
          jax.ShapeDtypeStruct((n, d2), jnp.float32),
          jax.ShapeDtypeStruct((n, 1), jnp.float32),
          jax.ShapeDtypeStruct((n, 1), jnp.float32),
          jax.ShapeDtypeStruct((8, 128), jnp.float32),
      ),
  )(parts, b, W, al, ar)


def _tc_finish(parts, dw, b):
  n = parts.shape[1]

  def body(p_ref, b_ref, o):
    s = p_ref[0] + p_ref[1]
    o[...] = s[:, :dw] / (s[:, dw:dw + 1] + 1e-9) + b_ref[...]

  return pl.pallas_call(
      body,
      out_shape=jax.ShapeDtypeStruct((n, dw), jnp.float32),
  )(parts, b)


def _make_sc_gat(D, DW):
  NG = -(-DW // 16)
  mesh = plsc.VectorSubcoreMesh(
      core_axis_name="c", subcore_axis_name="s", num_cores=NC, num_subcores=NS)

  @functools.partial(
      pl.kernel,
      out_type=jax.ShapeDtypeStruct((NC, N, D), jnp.float32),
      mesh=mesh,
      compiler_params=pltpu.CompilerParams(
          needs_layout_passes=False, use_tc_tiling_on_sc=False),
      scratch_types=[
          pltpu.VMEM((NCHUNK, CH), jnp.int32),
          pltpu.VMEM((NCHUNK, CH), jnp.int32),
          pltpu.VMEM((N,), jnp.float32),
          pltpu.VMEM((N,), jnp.float32),
          pltpu.VMEM((16,), jnp.float32),
          pltpu.VMEM((CH, D), jnp.float32),
          pltpu.VMEM((CH,), jnp.float32),
          pltpu.VMEM_SHARED((N, D), jnp.float32),
          pltpu.SemaphoreType.DMA,
      ],
  )
  def sc_gat(h_hbm, el_hbm, er_hbm, gm_hbm, srcm_hbm, dstm_hbm, zrow_hbm,
             out_hbm, src_v, dst_v, el_v, er_v, gm_v, rows_v, w_v,
             acc_sh, sem):
    c = lax.axis_index("c")
    s = lax.axis_index("s")
    wid = c * NS + s

    pltpu.sync_copy(zrow_hbm, acc_sh.at[pl.ds(s * SLAB, SLAB)])

    @pl.when(s == 0)
    def _zero_tail():
      pltpu.sync_copy(zrow_hbm.at[pl.ds(0, TAIL)],
                      acc_sh.at[pl.ds(NS * SLAB, TAIL)])

    pltpu.sync_copy(el_hbm, el_v)
    pltpu.sync_copy(er_hbm, er_v)
    pltpu.sync_copy(gm_hbm, gm_v)
    pltpu.sync_copy(srcm_hbm.at[wid], src_v)
    pltpu.sync_copy(dstm_hbm.at[wid], dst_v)
    plsc.subcore_barrier()

    def chunk_body(j, carry):
      pltpu.async_copy(h_hbm.at[src_v.at[j]], rows_v, sem).wait()

      def w_body(g, carry2):
        si = src_v[j, pl.ds(g * 16, 16)]
        di = dst_v[j, pl.ds(g * 16, 16)]
        els = plsc.load_gather(el_v, [si])
        erd = plsc.load_gather(er_v, [di])
        e = els + erd
        e = jnp.maximum(e, 0.2 * e)
        t = gm_v[...] + erd
        m = jnp.maximum(t, 0.2 * t)
        w = jnp.exp(e - m)
        pos = j * CH + g * 16 + lax.iota(jnp.int32, 16)
        w = jnp.where(pos < EPT, w, 0.0)
        w_v[pl.ds(g * 16, 16)] = w
        return carry2

      lax.fori_loop(0, CH // 16, w_body, 0, unroll=2)

      def scale_body(g, carry2):
        wg = w_v[pl.ds(g * 16, 16)]
        base = g * 16
        for k in range(16):
          wc = wg[k]
          i = base + k
          for q in range(NG):
            rows_v[i, pl.ds(q * 16, 16)] = rows_v[i, pl.ds(q * 16, 16)] * wc
        row16 = base + lax.iota(jnp.int32, 16)
        col16 = jnp.full((16,), DW, dtype=jnp.int32)
        plsc.store_scatter(rows_v, [row16, col16], wg)
        return carry2

      lax.fori_loop(0, CH // 16, scale_body, 0)

      pltpu.sync_copy(rows_v, acc_sh.at[dst_v.at[j]], add=True)
      return carry

    lax.fori_loop(0, NCHUNK, chunk_body, 0)

    plsc.subcore_barrier()
    pltpu.sync_copy(acc_sh.at[pl.ds(s * SLAB, SLAB)],
                    out_hbm.at[c, pl.ds(s * SLAB, SLAB)])

    @pl.when(s == 0)
    def _out_tail():
      pltpu.sync_copy(acc_sh.at[pl.ds(NS * SLAB, TAIL)],
                      out_hbm.at[c, pl.ds(NS * SLAB, TAIL)])

  return sc_gat


_sc_gat_l1 = _make_sc_gat(72, 64)
_sc_gat_l2 = _make_sc_gat(48, 40)


def kernel(in_feat, edge_index, W1, a_l1, a_r1, b1, W2, a_l2, a_r2, b2):
  f32 = jnp.float32
  src = edge_index[0]
  dst = edge_index[1]
  pad = EPT_PAD - EPT
  srcm = jnp.pad(src.reshape(NW, EPT), ((0, 0), (0, pad))).reshape(
      NW, NCHUNK, CH)
  dstm = jnp.pad(dst.reshape(NW, EPT), ((0, 0), (0, pad))).reshape(
      NW, NCHUNK, CH)

  W1a = jnp.pad(W1, ((0, 0), (0, 8)))
  al1 = jnp.pad(a_l1, (0, 8)).reshape(1, 72)
  ar1 = jnp.pad(a_r1, (0, 8)).reshape(1, 72)
  h1, el1, er1, gm1 = _tc_proj(in_feat, W1a, al1, ar1)
  gm1v = jnp.full((16,), gm1[0, 0], dtype=f32)
  z72 = jnp.zeros((SLAB, 72), dtype=f32)
  parts1 = _sc_gat_l1(h1, el1.reshape(N), er1.reshape(N), gm1v,
                      srcm, dstm, z72)

  W2a = jnp.pad(W2, ((0, 0), (0, 8)))
  al2 = jnp.pad(a_l2, (0, 8)).reshape(1, 48)
  ar2 = jnp.pad(a_r2, (0, 8)).reshape(1, 48)
  h2, el2, er2, gm2 = _tc_combine_proj(parts1, 64, b1.reshape(1, 64),
                                       W2a, al2, ar2)
  gm2v = jnp.full((16,), gm2[0, 0], dtype=f32)
  z48 = jnp.zeros((SLAB, 48), dtype=f32)
  parts2 = _sc_gat_l2(h2, el2.reshape(N), er2.reshape(N), gm2v,
                      srcm, dstm, z48)

  return _tc_finish(parts2, 40, b2.reshape(1, 40))
